# baseline (speedup 1.0000x reference)
import functools

import jax
import jax.numpy as jnp
from jax import lax
from jax.experimental import pallas as pl
from jax.experimental.pallas import tpu as pltpu
from jax.experimental.pallas import tpu_sc as plsc

VOCAB = 100000
D = 128
B = 4096
S = 50
C = 9
CP = 16
NW = 32
BPW = B // NW
NG = BPW // 2
GP = 2 * S
GPP = 104
NBUF = 4
SUB = 544
VCHUNK = 8 * SUB
NSTEP = 23
VP = NSTEP * VCHUNK


def _proj_body(t_ref, w_ref, o_ref):
    w = w_ref[...]
    o_ref[...] = jnp.concatenate(
        [jnp.dot(t_ref[pl.ds(SUB * j, SUB), :], w,
                 preferred_element_type=jnp.float32) for j in range(8)],
        axis=1)


def _project(table, wp):
    return pl.pallas_call(
        _proj_body,
        grid=(NSTEP,),
        in_specs=[
            pl.BlockSpec((VCHUNK, D), lambda i: (i, 0)),
            pl.BlockSpec((D, CP), lambda i: (0, 0)),
        ],
        out_specs=pl.BlockSpec((SUB, 8 * CP), lambda i: (i, 0)),
        out_shape=jax.ShapeDtypeStruct((VP // 8, 8 * CP), jnp.float32),
    )(table, wp)


_MESH = plsc.VectorSubcoreMesh(core_axis_name="c", subcore_axis_name="s")


@functools.partial(
    pl.kernel,
    mesh=_MESH,
    compiler_params=pltpu.CompilerParams(use_tc_tiling_on_sc=False),
    out_type=jax.ShapeDtypeStruct((B, CP), jnp.float32),
    scratch_types=[
        pltpu.VMEM((NG, GPP), jnp.int32),
        [pltpu.VMEM((GPP, CP), jnp.float32) for _ in range(NBUF)],
        pltpu.VMEM((BPW, CP), jnp.float32),
        pltpu.VMEM((CP,), jnp.float32),
        [[pltpu.SemaphoreType.DMA, pltpu.SemaphoreType.DMA]
         for _ in range(NBUF)],
    ],
)
def _pool(xg_hbm, p_hbm, bias_hbm, out_hbm,
          idx_v, rows_bufs, out_v, bias_v, sems):
    cid = lax.axis_index("c")
    sid = lax.axis_index("s")
    wid = sid * 2 + cid
    obase = wid * BPW

    pltpu.sync_copy(xg_hbm.at[wid], idx_v)
    pltpu.sync_copy(bias_hbm, bias_v)
    bvec = bias_v[...]

    H0 = 56

    def start_gather(g, rows_v, sem2):
        pltpu.async_copy(p_hbm.at[idx_v.at[g, pl.ds(0, H0)]],
                         rows_v.at[pl.ds(0, H0)], sem2[0])
        pltpu.async_copy(p_hbm.at[idx_v.at[g, pl.ds(H0, GPP - H0)]],
                         rows_v.at[pl.ds(H0, GPP - H0)], sem2[1])

    def wait_gather(g, rows_v, sem2):
        pltpu.make_async_copy(p_hbm.at[idx_v.at[g, pl.ds(0, H0)]],
                              rows_v.at[pl.ds(0, H0)], sem2[0]).wait()
        pltpu.make_async_copy(p_hbm.at[idx_v.at[g, pl.ds(H0, GPP - H0)]],
                              rows_v.at[pl.ds(H0, GPP - H0)], sem2[1]).wait()

    for b in range(NBUF):
        start_gather(b, rows_bufs[b], sems[b])

    def accumulate(rows_v, g):
        for r in range(2):
            p0 = bvec + rows_v[r * S, :]
            p1 = rows_v[r * S + 1, :]
            for k in range(2, S, 2):
                p0 = p0 + rows_v[r * S + k, :]
                p1 = p1 + rows_v[r * S + k + 1, :]
            out_v[2 * g + r, :] = p0 + p1

    def body(i, carry):
        for b in range(NBUF):
            g = NBUF * i + b
            rows_v, sem2 = rows_bufs[b], sems[b]
            wait_gather(g, rows_v, sem2)
            accumulate(rows_v, g)

            @pl.when(g + NBUF < NG)
            def _():
                start_gather(g + NBUF, rows_v, sem2)

        return carry

    lax.fori_loop(0, NG // NBUF, body, 0)

    pltpu.sync_copy(out_v, out_hbm.at[pl.ds(obase, BPW)])


def kernel(x, embed_table, W, b):
    wp = jnp.pad(W, ((0, 0), (0, CP - C))) * (1.0 / S)
    p = _project(embed_table, wp).reshape(VP, CP)
    rem = x % VCHUNK
    xr = ((x // VCHUNK) * SUB + rem % SUB) * 8 + rem // SUB
    xg = jnp.pad(xr.reshape(NW, NG, GP), ((0, 0), (0, 0), (0, GPP - GP)))
    b16 = jnp.pad(b, (0, CP - C))
    out16 = _pool(xg, p, b16)
    return out16[:, :C]


# speedup vs baseline: 1.4036x; 1.4036x over previous
import functools

import jax
import jax.numpy as jnp
from jax import lax
from jax.experimental import pallas as pl
from jax.experimental.pallas import tpu as pltpu
from jax.experimental.pallas import tpu_sc as plsc

VOCAB = 100000
D = 128
B = 4096
S = 50
C = 9
CP = 16
NW = 32
BPW = B // NW
NG = BPW // 2
GP = 2 * S
GPP = 104
NBUF = 4
SUB = 544
VCHUNK = 8 * SUB
NSTEP = 23
VP = NSTEP * VCHUNK


def _proj_body(t_ref, w_ref, o_ref):
    w = w_ref[...]
    o_ref[...] = jnp.concatenate(
        [jnp.dot(t_ref[pl.ds(SUB * j, SUB), :], w,
                 preferred_element_type=jnp.float32) for j in range(8)],
        axis=1)


def _project(table, wp):
    return pl.pallas_call(
        _proj_body,
        grid=(NSTEP,),
        in_specs=[
            pl.BlockSpec((VCHUNK, D), lambda i: (i, 0)),
            pl.BlockSpec((D, CP), lambda i: (0, 0)),
        ],
        out_specs=pl.BlockSpec((SUB, 8 * CP), lambda i: (i, 0)),
        out_shape=jax.ShapeDtypeStruct((VP // 8, 8 * CP), jnp.float32),
    )(table, wp)


_MESH = plsc.VectorSubcoreMesh(core_axis_name="c", subcore_axis_name="s")


@functools.partial(
    pl.kernel,
    mesh=_MESH,
    compiler_params=pltpu.CompilerParams(use_tc_tiling_on_sc=False),
    out_type=jax.ShapeDtypeStruct((B, CP), jnp.float32),
    scratch_types=[
        pltpu.VMEM((NG, GPP), jnp.int32),
        [pltpu.VMEM((GPP, CP), jnp.float32) for _ in range(NBUF)],
        pltpu.VMEM((BPW, CP), jnp.float32),
        pltpu.VMEM((CP,), jnp.float32),
        pltpu.VMEM_SHARED((VP, CP), jnp.float32),
        [pltpu.SemaphoreType.DMA for _ in range(NBUF)],
    ],
)
def _pool(xg_hbm, p_hbm, bias_hbm, out_hbm,
          idx_v, rows_bufs, out_v, bias_v, p_sh, sems):
    cid = lax.axis_index("c")
    sid = lax.axis_index("s")
    wid = sid * 2 + cid
    obase = wid * BPW
    SHR = VP // 16

    pltpu.sync_copy(p_hbm.at[pl.ds(sid * SHR, SHR)],
                    p_sh.at[pl.ds(sid * SHR, SHR)])
    pltpu.sync_copy(xg_hbm.at[wid], idx_v)
    pltpu.sync_copy(bias_hbm, bias_v)
    bvec = bias_v[...]
    plsc.subcore_barrier()

    for b in range(NBUF):
        pltpu.async_copy(p_sh.at[idx_v.at[b]], rows_bufs[b], sems[b])

    def accumulate(rows_v, g):
        for r in range(2):
            p0 = bvec + rows_v[r * S, :]
            p1 = rows_v[r * S + 1, :]
            for k in range(2, S, 2):
                p0 = p0 + rows_v[r * S + k, :]
                p1 = p1 + rows_v[r * S + k + 1, :]
            out_v[2 * g + r, :] = p0 + p1

    def body(i, carry):
        for b in range(NBUF):
            g = NBUF * i + b
            rows_v, sem = rows_bufs[b], sems[b]
            pltpu.make_async_copy(p_sh.at[idx_v.at[g]], rows_v, sem).wait()
            accumulate(rows_v, g)

            @pl.when(g + NBUF < NG)
            def _():
                pltpu.async_copy(p_sh.at[idx_v.at[g + NBUF]], rows_v, sem)

        return carry

    lax.fori_loop(0, NG // NBUF, body, 0)

    pltpu.sync_copy(out_v, out_hbm.at[pl.ds(obase, BPW)])


def kernel(x, embed_table, W, b):
    wp = jnp.pad(W, ((0, 0), (0, CP - C))) * (1.0 / S)
    p = _project(embed_table, wp).reshape(VP, CP)
    rem = x % VCHUNK
    xr = ((x // VCHUNK) * SUB + rem % SUB) * 8 + rem // SUB
    xg = jnp.pad(xr.reshape(NW, NG, GP), ((0, 0), (0, 0), (0, GPP - GP)))
    b16 = jnp.pad(b, (0, CP - C))
    out16 = _pool(xg, p, b16)
    return out16[:, :C]


# speedup vs baseline: 1.5283x; 1.0888x over previous
import functools

import jax
import jax.numpy as jnp
from jax import lax
from jax.experimental import pallas as pl
from jax.experimental.pallas import tpu as pltpu
from jax.experimental.pallas import tpu_sc as plsc

VOCAB = 100000
D = 128
B = 4096
S = 50
C = 9
CP = 16
NW = 32
BPW = B // NW
NG = BPW // 2
GP = 2 * S
GPP = 104
NBUF = 4
SUB = 1088
VCHUNK = 8 * SUB
NSTEP = 12
VP = NSTEP * VCHUNK


def _proj_body(t_ref, w_ref, o_ref):
    w = w_ref[...]
    o_ref[...] = jnp.concatenate(
        [jnp.dot(t_ref[pl.ds(SUB * j, SUB), :], w,
                 preferred_element_type=jnp.float32) for j in range(8)],
        axis=1)


def _project(table, wp):
    return pl.pallas_call(
        _proj_body,
        grid=(NSTEP,),
        in_specs=[
            pl.BlockSpec((VCHUNK, D), lambda i: (i, 0)),
            pl.BlockSpec((D, CP), lambda i: (0, 0)),
        ],
        out_specs=pl.BlockSpec((SUB, 8 * CP), lambda i: (i, 0)),
        out_shape=jax.ShapeDtypeStruct((VP // 8, 8 * CP), jnp.float32),
    )(table, wp)


_MESH = plsc.VectorSubcoreMesh(core_axis_name="c", subcore_axis_name="s")


@functools.partial(
    pl.kernel,
    mesh=_MESH,
    compiler_params=pltpu.CompilerParams(use_tc_tiling_on_sc=False),
    out_type=jax.ShapeDtypeStruct((B, CP), jnp.float32),
    scratch_types=[
        pltpu.VMEM((NG, GPP), jnp.int32),
        [pltpu.VMEM((GPP, CP), jnp.float32) for _ in range(NBUF)],
        pltpu.VMEM((BPW, CP), jnp.float32),
        pltpu.VMEM((CP,), jnp.float32),
        pltpu.VMEM_SHARED((VP, CP), jnp.float32),
        [pltpu.SemaphoreType.DMA for _ in range(NBUF)],
    ],
)
def _pool(xg_hbm, p_hbm, bias_hbm, out_hbm,
          idx_v, rows_bufs, out_v, bias_v, p_sh, sems):
    cid = lax.axis_index("c")
    sid = lax.axis_index("s")
    wid = sid * 2 + cid
    obase = wid * BPW
    SHR = VP // 16

    pltpu.sync_copy(p_hbm.at[pl.ds(sid * SHR, SHR)],
                    p_sh.at[pl.ds(sid * SHR, SHR)])
    pltpu.sync_copy(xg_hbm.at[wid], idx_v)
    pltpu.sync_copy(bias_hbm, bias_v)
    bvec = bias_v[...]
    plsc.subcore_barrier()

    for b in range(NBUF):
        pltpu.async_copy(p_sh.at[idx_v.at[b]], rows_bufs[b], sems[b])

    def accumulate(rows_v, g):
        for r in range(2):
            p0 = bvec + rows_v[r * S, :]
            p1 = rows_v[r * S + 1, :]
            for k in range(2, S, 2):
                p0 = p0 + rows_v[r * S + k, :]
                p1 = p1 + rows_v[r * S + k + 1, :]
            out_v[2 * g + r, :] = p0 + p1

    def body(i, carry):
        for b in range(NBUF):
            g = NBUF * i + b
            rows_v, sem = rows_bufs[b], sems[b]
            pltpu.make_async_copy(p_sh.at[idx_v.at[g]], rows_v, sem).wait()
            accumulate(rows_v, g)

            @pl.when(g + NBUF < NG)
            def _():
                pltpu.async_copy(p_sh.at[idx_v.at[g + NBUF]], rows_v, sem)

        return carry

    lax.fori_loop(0, NG // NBUF, body, 0)

    pltpu.sync_copy(out_v, out_hbm.at[pl.ds(obase, BPW)])


def kernel(x, embed_table, W, b):
    wp = jnp.pad(W, ((0, 0), (0, CP - C))) * (1.0 / S)
    p = _project(embed_table, wp).reshape(VP, CP)
    rem = x % VCHUNK
    xr = ((x // VCHUNK) * SUB + rem % SUB) * 8 + rem // SUB
    xg = jnp.pad(xr.reshape(NW, NG, GP), ((0, 0), (0, 0), (0, GPP - GP)))
    b16 = jnp.pad(b, (0, CP - C))
    out16 = _pool(xg, p, b16)
    return out16[:, :C]


# speedup vs baseline: 1.5965x; 1.0446x over previous
import functools

import jax
import jax.numpy as jnp
from jax import lax
from jax.experimental import pallas as pl
from jax.experimental.pallas import tpu as pltpu
from jax.experimental.pallas import tpu_sc as plsc

VOCAB = 100000
D = 128
B = 4096
S = 50
C = 9
CP = 16
NW = 32
BPW = B // NW
NG = BPW // 2
GP = 2 * S
GPP = 104
NBUF = 4
SUB = 2176
VCHUNK = 8 * SUB
NSTEP = 6
VP = NSTEP * VCHUNK


def _proj_body(t_ref, w_ref, o_ref):
    w = w_ref[...]
    o_ref[...] = jnp.concatenate(
        [jnp.dot(t_ref[pl.ds(SUB * j, SUB), :], w,
                 preferred_element_type=jnp.float32) for j in range(8)],
        axis=1)


def _project(table, wp):
    return pl.pallas_call(
        _proj_body,
        grid=(NSTEP,),
        in_specs=[
            pl.BlockSpec((VCHUNK, D), lambda i: (i, 0)),
            pl.BlockSpec((D, CP), lambda i: (0, 0)),
        ],
        out_specs=pl.BlockSpec((SUB, 8 * CP), lambda i: (i, 0)),
        out_shape=jax.ShapeDtypeStruct((VP // 8, 8 * CP), jnp.float32),
    )(table, wp)


_MESH = plsc.VectorSubcoreMesh(core_axis_name="c", subcore_axis_name="s")


@functools.partial(
    pl.kernel,
    mesh=_MESH,
    compiler_params=pltpu.CompilerParams(use_tc_tiling_on_sc=False),
    out_type=jax.ShapeDtypeStruct((B, CP), jnp.float32),
    scratch_types=[
        pltpu.VMEM((NG, GPP), jnp.int32),
        [pltpu.VMEM((GPP, CP), jnp.float32) for _ in range(NBUF)],
        pltpu.VMEM((BPW, CP), jnp.float32),
        pltpu.VMEM((CP,), jnp.float32),
        pltpu.VMEM_SHARED((VP, CP), jnp.float32),
        [pltpu.SemaphoreType.DMA for _ in range(NBUF)],
    ],
)
def _pool(xg_hbm, p_hbm, bias_hbm, out_hbm,
          idx_v, rows_bufs, out_v, bias_v, p_sh, sems):
    cid = lax.axis_index("c")
    sid = lax.axis_index("s")
    wid = sid * 2 + cid
    obase = wid * BPW
    SHR = VP // 16

    pltpu.sync_copy(p_hbm.at[pl.ds(sid * SHR, SHR)],
                    p_sh.at[pl.ds(sid * SHR, SHR)])
    pltpu.sync_copy(xg_hbm.at[wid], idx_v)
    pltpu.sync_copy(bias_hbm, bias_v)
    bvec = bias_v[...]
    plsc.subcore_barrier()

    for b in range(NBUF):
        pltpu.async_copy(p_sh.at[idx_v.at[b]], rows_bufs[b], sems[b])

    def accumulate(rows_v, g):
        for r in range(2):
            p0 = bvec + rows_v[r * S, :]
            p1 = rows_v[r * S + 1, :]
            for k in range(2, S, 2):
                p0 = p0 + rows_v[r * S + k, :]
                p1 = p1 + rows_v[r * S + k + 1, :]
            out_v[2 * g + r, :] = p0 + p1

    def body(i, carry):
        for b in range(NBUF):
            g = NBUF * i + b
            rows_v, sem = rows_bufs[b], sems[b]
            pltpu.make_async_copy(p_sh.at[idx_v.at[g]], rows_v, sem).wait()
            accumulate(rows_v, g)

            @pl.when(g + NBUF < NG)
            def _():
                pltpu.async_copy(p_sh.at[idx_v.at[g + NBUF]], rows_v, sem)

        return carry

    lax.fori_loop(0, NG // NBUF, body, 0)

    pltpu.sync_copy(out_v, out_hbm.at[pl.ds(obase, BPW)])


def kernel(x, embed_table, W, b):
    wp = jnp.pad(W, ((0, 0), (0, CP - C))) * (1.0 / S)
    p = _project(embed_table, wp).reshape(VP, CP)
    rem = x % VCHUNK
    xr = ((x // VCHUNK) * SUB + rem % SUB) * 8 + rem // SUB
    xg = jnp.pad(xr.reshape(NW, NG, GP), ((0, 0), (0, 0), (0, GPP - GP)))
    b16 = jnp.pad(b, (0, CP - C))
    out16 = _pool(xg, p, b16)
    return out16[:, :C]


# speedup vs baseline: 1.6038x; 1.0046x over previous
import functools

import jax
import jax.numpy as jnp
from jax import lax
from jax.experimental import pallas as pl
from jax.experimental.pallas import tpu as pltpu
from jax.experimental.pallas import tpu_sc as plsc

VOCAB = 100000
D = 128
B = 4096
S = 50
C = 9
CP = 16
NW = 32
BPW = B // NW
NG = BPW // 2
GP = 2 * S
GPP = 104
NBUF = 4
SUB = 3264
VCHUNK = 8 * SUB
NSTEP = 4
VP = NSTEP * VCHUNK


def _proj_body(t_ref, w_ref, o_ref):
    w = w_ref[...]
    o_ref[...] = jnp.concatenate(
        [jnp.dot(t_ref[pl.ds(SUB * j, SUB), :], w,
                 preferred_element_type=jnp.float32) for j in range(8)],
        axis=1)


def _project(table, wp):
    return pl.pallas_call(
        _proj_body,
        grid=(NSTEP,),
        in_specs=[
            pl.BlockSpec((VCHUNK, D), lambda i: (i, 0)),
            pl.BlockSpec((D, CP), lambda i: (0, 0)),
        ],
        out_specs=pl.BlockSpec((SUB, 8 * CP), lambda i: (i, 0)),
        out_shape=jax.ShapeDtypeStruct((VP // 8, 8 * CP), jnp.float32),
    )(table, wp)


_MESH = plsc.VectorSubcoreMesh(core_axis_name="c", subcore_axis_name="s")


@functools.partial(
    pl.kernel,
    mesh=_MESH,
    compiler_params=pltpu.CompilerParams(use_tc_tiling_on_sc=False),
    out_type=jax.ShapeDtypeStruct((B, CP), jnp.float32),
    scratch_types=[
        pltpu.VMEM((NG, GPP), jnp.int32),
        [pltpu.VMEM((GPP, CP), jnp.float32) for _ in range(NBUF)],
        pltpu.VMEM((BPW, CP), jnp.float32),
        pltpu.VMEM((CP,), jnp.float32),
        pltpu.VMEM_SHARED((VP, CP), jnp.float32),
        [pltpu.SemaphoreType.DMA for _ in range(NBUF)],
    ],
)
def _pool(xg_hbm, p_hbm, bias_hbm, out_hbm,
          idx_v, rows_bufs, out_v, bias_v, p_sh, sems):
    cid = lax.axis_index("c")
    sid = lax.axis_index("s")
    wid = sid * 2 + cid
    obase = wid * BPW
    SHR = VP // 16

    pltpu.sync_copy(p_hbm.at[pl.ds(sid * SHR, SHR)],
                    p_sh.at[pl.ds(sid * SHR, SHR)])
    pltpu.sync_copy(xg_hbm.at[wid], idx_v)
    pltpu.sync_copy(bias_hbm, bias_v)
    bvec = bias_v[...]
    plsc.subcore_barrier()

    for b in range(NBUF):
        pltpu.async_copy(p_sh.at[idx_v.at[b]], rows_bufs[b], sems[b])

    def accumulate(rows_v, g):
        for r in range(2):
            p0 = bvec + rows_v[r * S, :]
            p1 = rows_v[r * S + 1, :]
            for k in range(2, S, 2):
                p0 = p0 + rows_v[r * S + k, :]
                p1 = p1 + rows_v[r * S + k + 1, :]
            out_v[2 * g + r, :] = p0 + p1

    def body(i, carry):
        for b in range(NBUF):
            g = NBUF * i + b
            rows_v, sem = rows_bufs[b], sems[b]
            pltpu.make_async_copy(p_sh.at[idx_v.at[g]], rows_v, sem).wait()
            accumulate(rows_v, g)

            @pl.when(g + NBUF < NG)
            def _():
                pltpu.async_copy(p_sh.at[idx_v.at[g + NBUF]], rows_v, sem)

        return carry

    lax.fori_loop(0, NG // NBUF, body, 0)

    pltpu.sync_copy(out_v, out_hbm.at[pl.ds(obase, BPW)])


def kernel(x, embed_table, W, b):
    wp = jnp.pad(W, ((0, 0), (0, CP - C))) * (1.0 / S)
    p = _project(embed_table, wp).reshape(VP, CP)
    rem = x % VCHUNK
    xr = ((x // VCHUNK) * SUB + rem % SUB) * 8 + rem // SUB
    xg = jnp.pad(xr.reshape(NW, NG, GP), ((0, 0), (0, 0), (0, GPP - GP)))
    b16 = jnp.pad(b, (0, CP - C))
    out16 = _pool(xg, p, b16)
    return out16[:, :C]
